# Initial kernel scaffold; baseline (speedup 1.0000x reference)
import jax
import jax.numpy as jnp
from jax.experimental import pallas as pl


def kernel(species, elementnum_to_vector):
    raise NotImplementedError("write your pallas kernel here")


# speedup vs baseline: 1.9848x; 1.9848x over previous
import functools
import jax
import jax.numpy as jnp
from jax import lax
from jax.experimental import pallas as pl
from jax.experimental.pallas import tpu as pltpu
from jax.experimental.pallas import tpu_sc as plsc

_NUM_CORES = 2
_NUM_SUBCORES = 16
_NW = _NUM_CORES * _NUM_SUBCORES
_ALIGN = 8 * _NW


@functools.lru_cache(maxsize=None)
def _make_gather(num_rows_padded, table_rows, dim):
    b_per_w = num_rows_padded // _NW
    mesh = plsc.VectorSubcoreMesh(
        core_axis_name="c",
        subcore_axis_name="s",
        num_cores=_NUM_CORES,
        num_subcores=_NUM_SUBCORES,
    )

    @functools.partial(
        pl.kernel,
        out_type=jax.ShapeDtypeStruct((num_rows_padded, dim), jnp.float32),
        mesh=mesh,
        scratch_types=[
            pltpu.VMEM((b_per_w,), jnp.int32),
            pltpu.VMEM((b_per_w, dim), jnp.float32),
            pltpu.SemaphoreType.DMA,
        ],
        compiler_params=pltpu.CompilerParams(use_tc_tiling_on_sc=False),
    )
    def gather_kernel(idx_hbm, table_hbm, out_hbm, idx_v, rows_v, sem):
        wid = lax.axis_index("s") * _NUM_CORES + lax.axis_index("c")
        base = wid * b_per_w
        pltpu.sync_copy(idx_hbm.at[pl.ds(base, b_per_w)], idx_v)
        pltpu.async_copy(table_hbm.at[idx_v], rows_v, sem).wait()
        pltpu.sync_copy(rows_v, out_hbm.at[pl.ds(base, b_per_w)])

    return gather_kernel


def kernel(species, elementnum_to_vector):
    n = species.shape[0]
    table_rows, dim = elementnum_to_vector.shape
    n_pad = ((n + _ALIGN - 1) // _ALIGN) * _ALIGN
    sp = jnp.pad(species, (0, n_pad - n)) if n_pad != n else species
    out = _make_gather(n_pad, table_rows, dim)(sp, elementnum_to_vector)
    return out[:n]


# speedup vs baseline: 2.5614x; 1.2906x over previous
import functools
import jax
import jax.numpy as jnp
from jax import lax
from jax.experimental import pallas as pl
from jax.experimental.pallas import tpu as pltpu
from jax.experimental.pallas import tpu_sc as plsc

_NUM_CORES = 2
_NUM_SUBCORES = 16
_NW = _NUM_CORES * _NUM_SUBCORES
_ALIGN = 8 * _NW


@functools.lru_cache(maxsize=None)
def _make_gather(num_rows_padded, table_rows, dim):
    b_per_w = num_rows_padded // _NW
    mesh = plsc.VectorSubcoreMesh(
        core_axis_name="c",
        subcore_axis_name="s",
        num_cores=_NUM_CORES,
        num_subcores=_NUM_SUBCORES,
    )

    @functools.partial(
        pl.kernel,
        out_type=jax.ShapeDtypeStruct((num_rows_padded, dim), jnp.float32),
        mesh=mesh,
        scratch_types=[
            pltpu.VMEM((b_per_w,), jnp.int32),
            pltpu.VMEM((b_per_w, dim), jnp.float32),
            pltpu.VMEM_SHARED((table_rows, dim), jnp.float32),
            pltpu.SemaphoreType.DMA,
        ],
        compiler_params=pltpu.CompilerParams(use_tc_tiling_on_sc=False),
    )
    def gather_kernel(idx_hbm, table_hbm, out_hbm, idx_v, rows_v, table_s, sem):
        sid = lax.axis_index("s")
        wid = sid * _NUM_CORES + lax.axis_index("c")
        base = wid * b_per_w

        @pl.when(sid == 0)
        def _stage_table():
            pltpu.sync_copy(table_hbm, table_s)

        pltpu.sync_copy(idx_hbm.at[pl.ds(base, b_per_w)], idx_v)
        plsc.subcore_barrier()
        pltpu.async_copy(table_s.at[idx_v], rows_v, sem).wait()
        pltpu.sync_copy(rows_v, out_hbm.at[pl.ds(base, b_per_w)])

    return gather_kernel


def kernel(species, elementnum_to_vector):
    n = species.shape[0]
    table_rows, dim = elementnum_to_vector.shape
    n_pad = ((n + _ALIGN - 1) // _ALIGN) * _ALIGN
    sp = jnp.pad(species, (0, n_pad - n)) if n_pad != n else species
    out = _make_gather(n_pad, table_rows, dim)(sp, elementnum_to_vector)
    return out[:n]


# speedup vs baseline: 3.7318x; 1.4569x over previous
import functools
import jax
import jax.numpy as jnp
from jax import lax
from jax.experimental import pallas as pl
from jax.experimental.pallas import tpu as pltpu
from jax.experimental.pallas import tpu_sc as plsc

_NUM_CORES = 2
_NUM_SUBCORES = 16
_NW = _NUM_CORES * _NUM_SUBCORES


@functools.lru_cache(maxsize=None)
def _make_gather(n, table_rows, dim):
    chunk = ((n + _NW - 1) // _NW + 7) // 8 * 8
    assert n % 8 == 0 and n >= chunk

    mesh = plsc.VectorSubcoreMesh(
        core_axis_name="c",
        subcore_axis_name="s",
        num_cores=_NUM_CORES,
        num_subcores=_NUM_SUBCORES,
    )

    @functools.partial(
        pl.kernel,
        out_type=jax.ShapeDtypeStruct((n, dim), jnp.float32),
        mesh=mesh,
        scratch_types=[
            pltpu.VMEM((chunk,), jnp.int32),
            pltpu.VMEM((chunk, dim), jnp.float32),
            pltpu.VMEM_SHARED((table_rows, dim), jnp.float32),
            pltpu.SemaphoreType.DMA,
        ],
        compiler_params=pltpu.CompilerParams(use_tc_tiling_on_sc=False),
    )
    def gather_kernel(idx_hbm, table_hbm, out_hbm, idx_v, rows_v, table_s, sem):
        sid = lax.axis_index("s")
        wid = sid * _NUM_CORES + lax.axis_index("c")
        base = jnp.minimum(wid * chunk, n - chunk)

        @pl.when(sid == 0)
        def _stage_table():
            pltpu.sync_copy(table_hbm, table_s)

        pltpu.sync_copy(idx_hbm.at[pl.ds(base, chunk)], idx_v)
        plsc.subcore_barrier()
        pltpu.async_copy(table_s.at[idx_v], rows_v, sem).wait()
        pltpu.sync_copy(rows_v, out_hbm.at[pl.ds(base, chunk)])

    return gather_kernel


def kernel(species, elementnum_to_vector):
    n = species.shape[0]
    table_rows, dim = elementnum_to_vector.shape
    return _make_gather(n, table_rows, dim)(species, elementnum_to_vector)


# speedup vs baseline: 8.8452x; 2.3702x over previous
import functools
import jax
import jax.numpy as jnp
from jax import lax
from jax.experimental import pallas as pl
from jax.experimental.pallas import tpu as pltpu
from jax.experimental.pallas import tpu_sc as plsc

_NUM_CORES = 2
_NUM_SUBCORES = 16
_NW = _NUM_CORES * _NUM_SUBCORES
_LANES = 16
_SPAD = 128


@functools.lru_cache(maxsize=None)
def _make_gather(n, dim):
    assert dim == 16 and n % 16 == 0
    n_pad = (n + 127) // 128 * 128
    t0_total = n_pad // 128
    cnt_base, cnt_rem = divmod(t0_total, _NW)
    max_cnt = cnt_base + (1 if cnt_rem else 0)
    max_rows = max_cnt * 128
    assert n >= max_rows

    mesh = plsc.VectorSubcoreMesh(
        core_axis_name="c",
        subcore_axis_name="s",
        num_cores=_NUM_CORES,
        num_subcores=_NUM_SUBCORES,
    )

    @functools.partial(
        pl.kernel,
        out_type=jax.ShapeDtypeStruct((2, t0_total, 8, 128), jnp.float32),
        mesh=mesh,
        scratch_types=[
            pltpu.VMEM((_SPAD * dim * _LANES,), jnp.float32),
            pltpu.VMEM((max_rows,), jnp.int32),
            pltpu.VMEM((2, max_cnt, 8, 128), jnp.float32),
            pltpu.SemaphoreType.DMA,
        ],
        compiler_params=pltpu.CompilerParams(
            use_tc_tiling_on_sc=False,
            needs_layout_passes=False,
        ),
    )
    def gather_kernel(idx_hbm, rep_hbm, out_hbm, rep_v, sidx_v, buf, sem):
        wid = lax.axis_index("s") * _NUM_CORES + lax.axis_index("c")
        start = wid * cnt_base + jnp.minimum(wid, cnt_rem)
        cnt = cnt_base + jnp.where(wid < cnt_rem, 1, 0)

        stage_base = jnp.minimum(start * 128, n - max_rows)
        off = start * 128 - stage_base
        rep_dma = pltpu.async_copy(rep_hbm, rep_v, sem)
        idx_dma = pltpu.async_copy(
            idx_hbm.at[pl.ds(stage_base, max_rows)], sidx_v, sem
        )
        rep_dma.wait()
        idx_dma.wait()

        lane = lax.iota(jnp.int32, _LANES)
        col_off = [lane + c * _SPAD * _LANES for c in range(dim)]

        @plsc.parallel_loop(0, cnt)
        def per_tile_col(g):
            row0 = off + g * 128
            for j in range(8):
                s16 = sidx_v[pl.ds(jnp.minimum(row0 + j * 16, max_rows - 16), 16)]
                base16 = s16 * _LANES
                for c in range(dim):
                    vals = plsc.load_gather(rep_v, [base16 + col_off[c]])
                    buf[c // 8, g, c % 8, pl.ds(j * 16, 16)] = vals
            pltpu.async_copy(
                buf.at[:, pl.ds(g, 1)], out_hbm.at[:, pl.ds(start + g, 1)], sem
            )

        def drain(g, carry):
            pltpu.make_async_copy(
                buf.at[:, pl.ds(0, 1)], out_hbm.at[:, pl.ds(start, 1)], sem
            ).wait()
            return carry

        lax.fori_loop(0, cnt, drain, 0)

    return gather_kernel


def kernel(species, elementnum_to_vector):
    n = species.shape[0]
    table_rows, dim = elementnum_to_vector.shape
    n_pad = (n + 127) // 128 * 128
    tbl_t = jnp.pad(elementnum_to_vector.T, ((0, 0), (0, _SPAD - table_rows)))
    rep = jnp.repeat(tbl_t.reshape(-1), _LANES)
    x4 = _make_gather(n, dim)(species, rep)
    return x4.transpose(1, 3, 0, 2).reshape(n_pad, dim)[:n]
